# baseline (speedup 1.0000x reference)
import jax
import jax.numpy as jnp
from jax import lax
from jax.experimental import pallas as pl
from jax.experimental.pallas import tpu as pltpu
from jax.experimental.pallas import tpu_sc as plsc

_D = 768
_SL = 6
_LN = 128
_L = 50
_LP = 56
_B = 4096
_NC = 2
_NS = 16
_NW = _NC * _NS
_BPW = _B // _NW


def _pooled_row(rows_v, acc_v):
    chunks = [(j, c) for j in range(_SL) for c in range(_LN // 16)]
    init = tuple(rows_v[0, j, pl.ds(c * 16, 16)] for j, c in chunks)

    def add_row(l, accs):
        return tuple(accs[i] + rows_v[l, j, pl.ds(c * 16, 16)]
                     for i, (j, c) in enumerate(chunks))

    accs = lax.fori_loop(1, _L, add_row, init)
    scale = jnp.float32(1.0 / _L)
    for i, (j, c) in enumerate(chunks):
        acc_v[pl.ds(j * _LN + c * 16, 16)] = accs[i] * scale


def _body(ids_hbm, table_hbm, out_hbm,
          ids_v, rows_a, rows_b, acc_a, acc_b,
          gsem_a, gsem_b, osem_a, osem_b):
    wid = lax.axis_index("s") * _NC + lax.axis_index("c")
    base = wid * _BPW

    pltpu.sync_copy(ids_hbm.at[pl.ds(base * _LP, _BPW * _LP)], ids_v)

    _PIECES = ((0, 48), (48, 8))

    def gather_row(b, rows_v, gsem):
        for s, n in _PIECES:
            pltpu.async_copy(
                table_hbm.at[ids_v.at[pl.ds(b * _LP + s, n)]],
                rows_v.at[pl.ds(s, n)], gsem)

    def wait_row(b, rows_v, gsem):
        for s, n in _PIECES:
            pltpu.make_async_copy(
                table_hbm.at[ids_v.at[pl.ds(b * _LP + s, n)]],
                rows_v.at[pl.ds(s, n)], gsem).wait()

    gather_row(0, rows_a, gsem_a)
    gather_row(1, rows_b, gsem_b)

    def pair(p, _):
        b0 = 2 * p
        for off, rows_v, acc_v, gsem, osem in (
                (0, rows_a, acc_a, gsem_a, osem_a),
                (1, rows_b, acc_b, gsem_b, osem_b)):
            b = b0 + off
            wait_row(b, rows_v, gsem)
            @pl.when(b >= 2)
            def _():
                pltpu.make_async_copy(
                    acc_v, out_hbm.at[base + b - 2], osem).wait()
            _pooled_row(rows_v, acc_v)
            pltpu.async_copy(acc_v, out_hbm.at[base + b], osem)
            @pl.when(b + 2 < _BPW)
            def _():
                gather_row(b + 2, rows_v, gsem)
        return 0

    lax.fori_loop(0, _BPW // 2, pair, 0)

    pltpu.make_async_copy(acc_a, out_hbm.at[base + _BPW - 2], osem_a).wait()
    pltpu.make_async_copy(acc_b, out_hbm.at[base + _BPW - 1], osem_b).wait()


_mesh = plsc.VectorSubcoreMesh(core_axis_name="c", subcore_axis_name="s")

_sc_call = pl.kernel(
    _body,
    out_type=jax.ShapeDtypeStruct((_B, _D), jnp.float32),
    mesh=_mesh,
    scratch_types=[
        pltpu.VMEM((_BPW * _LP,), jnp.int32),
        pltpu.VMEM((_LP, _SL, _LN), jnp.float32),
        pltpu.VMEM((_LP, _SL, _LN), jnp.float32),
        pltpu.VMEM((_D,), jnp.float32),
        pltpu.VMEM((_D,), jnp.float32),
        pltpu.SemaphoreType.DMA,
        pltpu.SemaphoreType.DMA,
        pltpu.SemaphoreType.DMA,
        pltpu.SemaphoreType.DMA,
    ],
)


@jax.jit
def _run(input_ids, table):
    ids_flat = jnp.pad(input_ids, ((0, 0), (0, _LP - _L))).reshape(-1)
    table3 = table.reshape(table.shape[0], _SL, _LN)
    return _sc_call(ids_flat, table3)


def kernel(input_ids, table):
    return _run(input_ids, table)


# speedup vs baseline: 1.1275x; 1.1275x over previous
import jax
import jax.numpy as jnp
from jax import lax
from jax.experimental import pallas as pl
from jax.experimental.pallas import tpu as pltpu
from jax.experimental.pallas import tpu_sc as plsc

_D = 768
_L = 50
_LP = 56
_B = 4096
_NC = 2
_NS = 16
_NW = _NC * _NS
_BPW = _B // _NW
_CHUNKS = _D // 16

_PIECES = ((0, 16), (16, 16), (32, 16), (48, 8))


def _pooled_row(rows_v, acc_v):
    init = tuple(rows_v[0, pl.ds(c * 16, 16)] for c in range(_CHUNKS))

    def add_row(l, accs):
        return tuple(accs[c] + rows_v[l, pl.ds(c * 16, 16)]
                     for c in range(_CHUNKS))

    accs = lax.fori_loop(1, _L, add_row, init)
    scale = jnp.float32(1.0 / _L)
    for c in range(_CHUNKS):
        acc_v[pl.ds(c * 16, 16)] = accs[c] * scale


def _body(ids_hbm, table_hbm, out_hbm,
          ids_v, rows_a, rows_b, acc_a, acc_b,
          gsa0, gsa1, gsa2, gsa3, gsb0, gsb1, gsb2, gsb3,
          osem_a, osem_b):
    wid = lax.axis_index("s") * _NC + lax.axis_index("c")
    base = wid * _BPW
    gsems_a = (gsa0, gsa1, gsa2, gsa3)
    gsems_b = (gsb0, gsb1, gsb2, gsb3)

    pltpu.sync_copy(ids_hbm.at[pl.ds(base * _LP, _BPW * _LP)], ids_v)

    def gather_row(b, rows_v, gsems):
        for (s, n), gsem in zip(_PIECES, gsems):
            pltpu.async_copy(
                table_hbm.at[ids_v.at[pl.ds(b * _LP + s, n)]],
                rows_v.at[pl.ds(s, n)], gsem)

    def wait_row(b, rows_v, gsems):
        for (s, n), gsem in zip(_PIECES, gsems):
            pltpu.make_async_copy(
                table_hbm.at[ids_v.at[pl.ds(b * _LP + s, n)]],
                rows_v.at[pl.ds(s, n)], gsem).wait()

    gather_row(0, rows_a, gsems_a)
    gather_row(1, rows_b, gsems_b)

    def pair(p, _):
        b0 = 2 * p
        for off, rows_v, acc_v, gsems, osem in (
                (0, rows_a, acc_a, gsems_a, osem_a),
                (1, rows_b, acc_b, gsems_b, osem_b)):
            b = b0 + off
            wait_row(b, rows_v, gsems)
            @pl.when(b >= 2)
            def _():
                pltpu.make_async_copy(
                    acc_v, out_hbm.at[base + b - 2], osem).wait()
            _pooled_row(rows_v, acc_v)
            pltpu.async_copy(acc_v, out_hbm.at[base + b], osem)
            @pl.when(b + 2 < _BPW)
            def _():
                gather_row(b + 2, rows_v, gsems)
        return 0

    lax.fori_loop(0, _BPW // 2, pair, 0)

    pltpu.make_async_copy(acc_a, out_hbm.at[base + _BPW - 2], osem_a).wait()
    pltpu.make_async_copy(acc_b, out_hbm.at[base + _BPW - 1], osem_b).wait()


_mesh = plsc.VectorSubcoreMesh(core_axis_name="c", subcore_axis_name="s")

_sc_call = pl.kernel(
    _body,
    out_type=jax.ShapeDtypeStruct((_B, _D), jnp.float32),
    mesh=_mesh,
    scratch_types=[
        pltpu.VMEM((_BPW * _LP,), jnp.int32),
        pltpu.VMEM((_LP, _D), jnp.float32),
        pltpu.VMEM((_LP, _D), jnp.float32),
        pltpu.VMEM((_D,), jnp.float32),
        pltpu.VMEM((_D,), jnp.float32),
    ] + [pltpu.SemaphoreType.DMA] * 10,
)


@jax.jit
def _run(input_ids, table):
    ids_flat = jnp.pad(input_ids, ((0, 0), (0, _LP - _L))).reshape(-1)
    return _sc_call(ids_flat, table)


def kernel(input_ids, table):
    return _run(input_ids, table)


# speedup vs baseline: 4.8746x; 4.3233x over previous
import jax
import jax.numpy as jnp
from jax import lax
from jax.experimental import pallas as pl
from jax.experimental.pallas import tpu as pltpu
from jax.experimental.pallas import tpu_sc as plsc

_D = 768
_L = 50
_LM = 48
_GR = 8
_GT = _GR * 2
_B = 4096
_NC = 2
_NS = 16
_NW = _NC * _NS
_BPW = _B // _NW
_GPW = _BPW // _GR
_CHUNKS = _D // 16


def _pooled_row(rows_v, tail_v, t0, acc_v):
    init = tuple(rows_v[0, pl.ds(c * 16, 16)] for c in range(_CHUNKS))

    def add_row(l, accs):
        return tuple(accs[c] + rows_v[l, pl.ds(c * 16, 16)]
                     for c in range(_CHUNKS))

    accs = lax.fori_loop(1, _LM, add_row, init)
    accs = tuple(accs[c] + tail_v[t0, pl.ds(c * 16, 16)]
                 for c in range(_CHUNKS))
    accs = tuple(accs[c] + tail_v[t0 + 1, pl.ds(c * 16, 16)]
                 for c in range(_CHUNKS))
    scale = jnp.float32(1.0 / _L)
    for c in range(_CHUNKS):
        acc_v[pl.ds(c * 16, 16)] = accs[c] * scale


def _body(idm_hbm, idt_hbm, table_hbm, out_hbm,
          idm_v, idt_v, rows_a, rows_b, tail_a, tail_b, acc_a, acc_b,
          gsem_a, gsem_b, tsem_a, tsem_b, osem_a, osem_b):
    wid = lax.axis_index("s") * _NC + lax.axis_index("c")
    base = wid * _BPW

    pltpu.sync_copy(idm_hbm.at[pl.ds(base * _LM, _BPW * _LM)], idm_v)
    pltpu.sync_copy(idt_hbm.at[pl.ds(wid * _GPW * _GT, _GPW * _GT)], idt_v)

    def gather_row(b, rows_v, gsem):
        pltpu.async_copy(
            table_hbm.at[idm_v.at[pl.ds(b * _LM, _LM)]], rows_v, gsem)

    def wait_row(b, rows_v, gsem):
        pltpu.make_async_copy(
            table_hbm.at[idm_v.at[pl.ds(b * _LM, _LM)]], rows_v, gsem).wait()

    def gather_tail(g, tail_v, tsem):
        pltpu.async_copy(
            table_hbm.at[idt_v.at[pl.ds(g * _GT, _GT)]], tail_v, tsem)

    def wait_tail(g, tail_v, tsem):
        pltpu.make_async_copy(
            table_hbm.at[idt_v.at[pl.ds(g * _GT, _GT)]], tail_v, tsem).wait()

    gather_tail(0, tail_a, tsem_a)
    gather_tail(1, tail_b, tsem_b)
    gather_row(0, rows_a, gsem_a)
    gather_row(1, rows_b, gsem_b)

    def group_pair(gp, _):
        for tg, tail_v, tsem in ((0, tail_a, tsem_a), (1, tail_b, tsem_b)):
            g = 2 * gp + tg
            wait_tail(g, tail_v, tsem)

            def row_pair(rp, _):
                for off, rows_v, acc_v, gsem, osem in (
                        (0, rows_a, acc_a, gsem_a, osem_a),
                        (1, rows_b, acc_b, gsem_b, osem_b)):
                    r = 2 * rp + off
                    b = g * _GR + r
                    wait_row(b, rows_v, gsem)
                    @pl.when(b >= 2)
                    def _():
                        pltpu.make_async_copy(
                            acc_v, out_hbm.at[base + b - 2], osem).wait()
                    _pooled_row(rows_v, tail_v, r * 2, acc_v)
                    pltpu.async_copy(acc_v, out_hbm.at[base + b], osem)
                    @pl.when(b + 2 < _BPW)
                    def _():
                        gather_row(b + 2, rows_v, gsem)
                return 0

            lax.fori_loop(0, _GR // 2, row_pair, 0)

            @pl.when(g + 2 < _GPW)
            def _():
                gather_tail(g + 2, tail_v, tsem)
        return 0

    lax.fori_loop(0, _GPW // 2, group_pair, 0)

    pltpu.make_async_copy(acc_a, out_hbm.at[base + _BPW - 2], osem_a).wait()
    pltpu.make_async_copy(acc_b, out_hbm.at[base + _BPW - 1], osem_b).wait()


_mesh = plsc.VectorSubcoreMesh(core_axis_name="c", subcore_axis_name="s")

_sc_call = pl.kernel(
    _body,
    out_type=jax.ShapeDtypeStruct((_B, _D), jnp.float32),
    scratch_types=[
        pltpu.VMEM((_BPW * _LM,), jnp.int32),
        pltpu.VMEM((_GPW * _GT,), jnp.int32),
        pltpu.VMEM((_LM, _D), jnp.float32),
        pltpu.VMEM((_LM, _D), jnp.float32),
        pltpu.VMEM((_GT, _D), jnp.float32),
        pltpu.VMEM((_GT, _D), jnp.float32),
        pltpu.VMEM((_D,), jnp.float32),
        pltpu.VMEM((_D,), jnp.float32),
    ] + [pltpu.SemaphoreType.DMA] * 6,
    mesh=_mesh,
)


@jax.jit
def _run(input_ids, table):
    ids_main = input_ids[:, :_LM].reshape(-1)
    ids_tail = input_ids[:, _LM:].reshape(-1)
    return _sc_call(ids_main, ids_tail, table)


def kernel(input_ids, table):
    return _run(input_ids, table)
